# Initial kernel scaffold; baseline (speedup 1.0000x reference)
import functools

import jax
import jax.numpy as jnp
from jax import lax
from jax.experimental import pallas as pl
from jax.experimental.pallas import tpu as pltpu
from jax.experimental.pallas import tpu_sc as plsc


def _table_body(nh_ref, se_ref, w1b_ref, w1d_ref, t_ref):
    h = nh_ref.shape[1]
    t_ref[:, :h] = jnp.dot(nh_ref[:], w1b_ref[:], preferred_element_type=jnp.float32)
    t_ref[:, h:] = jnp.dot(se_ref[:], w1d_ref[:], preferred_element_type=jnp.float32)


def _make_table(node_h, seq_emb, w1b, w1d, blk=1000):
    res, h = node_h.shape
    nb = res // blk
    return pl.pallas_call(
        _table_body,
        grid=(nb,),
        in_specs=[
            pl.BlockSpec((blk, h), lambda i: (i, 0)),
            pl.BlockSpec((blk, h), lambda i: (i, 0)),
            pl.BlockSpec((h, h), lambda i: (0, 0)),
            pl.BlockSpec((h, h), lambda i: (0, 0)),
        ],
        out_specs=pl.BlockSpec((blk, 2 * h), lambda i: (i, 0)),
        out_shape=jax.ShapeDtypeStruct((res, 2 * h), jnp.float32),
    )(node_h, seq_emb, w1b, w1d)


def _sc_gather(table, flat_idx, chunk=80):
    res, d = table.shape
    e = flat_idx.shape[0]
    info = plsc.get_sparse_core_info()
    nc, ns = info.num_cores, info.num_subcores
    nw = nc * ns
    epw = e // nw
    nchunks = epw // chunk
    mesh = plsc.VectorSubcoreMesh(core_axis_name="c", subcore_axis_name="s")

    @functools.partial(
        pl.kernel,
        out_type=jax.ShapeDtypeStruct((e, d), jnp.float32),
        mesh=mesh,
        scratch_types=[
            pltpu.VMEM((chunk,), jnp.int32),
            pltpu.VMEM((chunk, d), jnp.float32),
            pltpu.SemaphoreType.DMA,
        ],
    )
    def gather_k(t_hbm, idx_hbm, out_hbm, idx_v, rows_v, sem):
        wid = lax.axis_index("s") * nc + lax.axis_index("c")
        base = wid * epw

        def body(i, carry):
            off = base + i * chunk
            pltpu.sync_copy(idx_hbm.at[pl.ds(off, chunk)], idx_v)
            pltpu.async_copy(t_hbm.at[idx_v], rows_v, sem).wait()
            pltpu.sync_copy(rows_v, out_hbm.at[pl.ds(off, chunk)])
            return carry

        lax.fori_loop(0, nchunks, body, 0)

    return gather_k(table, flat_idx)


def _ln(x, g, b, eps=1e-5):
    mu = jnp.mean(x, axis=-1, keepdims=True)
    var = jnp.mean((x - mu) ** 2, axis=-1, keepdims=True)
    return (x - mu) / jnp.sqrt(var + eps) * g + b


def _gelu(x):
    return jax.nn.gelu(x, approximate=False)


def _main_body(nh_ref, eh_ref, tg_ref, ar_ref, w1a, b1r, w1c, w2, b2r, w3, b3r,
               wf1, bf1r, wf2, bf2r, g1r, be1r, g2r, be2r, out_ref):
    blk, k, h = eh_ref.shape
    f32 = jnp.float32
    nh = nh_ref[:]
    a = jnp.dot(nh, w1a[:], preferred_element_type=f32) + b1r[:]
    eh = eh_ref[:].reshape(blk * k, h)
    pre = jnp.dot(eh, w1c[:], preferred_element_type=f32).reshape(blk, k, h)
    tg = tg_ref[:]
    bg = tg[:, :, :h]
    dg = tg[:, :, h:]
    arb = lax.broadcast_in_dim(ar_ref[:], (blk, k, h), (0, 1))
    x = pre + a[:, None, :] + bg + arb * dg
    x = _gelu(x)
    x2 = _gelu(jnp.dot(x.reshape(blk * k, h), w2[:], preferred_element_type=f32) + b2r[:])
    s = jnp.sum(x2.reshape(blk, k, h), axis=1)
    agg = jnp.dot(s, w3[:], preferred_element_type=f32) + k * b3r[:]
    nh1 = _ln(nh + agg, g1r[:], be1r[:])
    ff1 = _gelu(jnp.dot(nh1, wf1[:], preferred_element_type=f32) + bf1r[:])
    ff = jnp.dot(ff1, wf2[:], preferred_element_type=f32) + bf2r[:]
    out_ref[:] = _ln(nh1 + ff, g2r[:], be2r[:])


def _main(node_h, edge_h, tg3, ar_mask, w1a, b1r, w1c, w2, b2r, w3, b3r,
          wf1, bf1r, wf2, bf2r, g1r, be1r, g2r, be2r, blk=250, interpret=False):
    res, k, h = edge_h.shape
    nb = res // blk
    full = lambda s: pl.BlockSpec(s, lambda i: tuple(0 for _ in s))
    return pl.pallas_call(
        _main_body,
        grid=(nb,),
        in_specs=[
            pl.BlockSpec((blk, h), lambda i: (i, 0)),
            pl.BlockSpec((blk, k, h), lambda i: (i, 0, 0)),
            pl.BlockSpec((blk, k, 2 * h), lambda i: (i, 0, 0)),
            pl.BlockSpec((blk, k), lambda i: (i, 0)),
            full((h, h)), full((1, h)),
            full((h, h)), full((h, h)), full((1, h)),
            full((h, h)), full((1, h)),
            full((h, 4 * h)), full((1, 4 * h)),
            full((4 * h, h)), full((1, h)),
            full((1, h)), full((1, h)), full((1, h)), full((1, h)),
        ],
        out_specs=pl.BlockSpec((blk, h), lambda i: (i, 0)),
        out_shape=jax.ShapeDtypeStruct((res, h), jnp.float32),
        interpret=interpret,
    )(node_h, edge_h, tg3, ar_mask, w1a, b1r, w1c, w2, b2r, w3, b3r,
      wf1, bf1r, wf2, bf2r, g1r, be1r, g2r, be2r)


def kernel(node_h, edge_h, edge_idx, seq_emb, ar_mask, W1, b1, W2, b2, W3, b3,
           Wf1, bf1, Wf2, bf2, g1, be1, g2, be2):
    res, h = node_h.shape
    k = edge_idx.shape[1]
    w1a, w1b, w1c, w1d = W1[:h], W1[h:2 * h], W1[2 * h:3 * h], W1[3 * h:]
    table = _make_table(node_h, seq_emb, w1b, w1d)
    tg = _sc_gather(table, edge_idx.reshape(-1))
    r1 = lambda v: v.reshape(1, -1)
    return _main(node_h, edge_h, tg.reshape(res, k, 2 * h), ar_mask,
                 w1a, r1(b1), w1c, W2, r1(b2), W3, r1(b3),
                 Wf1, r1(bf1), Wf2, r1(bf2), r1(g1), r1(be1), r1(g2), r1(be2))


# speedup vs baseline: 4.6942x; 4.6942x over previous
import functools

import jax
import jax.numpy as jnp
from jax import lax
from jax.experimental import pallas as pl
from jax.experimental.pallas import tpu as pltpu
from jax.experimental.pallas import tpu_sc as plsc


def _table_body(nh_ref, se_ref, w1b_ref, w1d_ref, t_ref):
    h = nh_ref.shape[1]
    t_ref[:, :h] = jnp.dot(nh_ref[:], w1b_ref[:], preferred_element_type=jnp.float32)
    t_ref[:, h:] = jnp.dot(se_ref[:], w1d_ref[:], preferred_element_type=jnp.float32)


def _make_table(node_h, seq_emb, w1b, w1d, blk=1000):
    res, h = node_h.shape
    nb = res // blk
    return pl.pallas_call(
        _table_body,
        grid=(nb,),
        in_specs=[
            pl.BlockSpec((blk, h), lambda i: (i, 0)),
            pl.BlockSpec((blk, h), lambda i: (i, 0)),
            pl.BlockSpec((h, h), lambda i: (0, 0)),
            pl.BlockSpec((h, h), lambda i: (0, 0)),
        ],
        out_specs=pl.BlockSpec((blk, 2 * h), lambda i: (i, 0)),
        out_shape=jax.ShapeDtypeStruct((res, 2 * h), jnp.float32),
    )(node_h, seq_emb, w1b, w1d)


def _sc_gather(table, flat_idx, chunk=80):
    res, d = table.shape
    e = flat_idx.shape[0]
    info = plsc.get_sparse_core_info()
    nc, ns = info.num_cores, info.num_subcores
    nw = nc * ns
    epw = e // nw
    nchunks = epw // chunk
    mesh = plsc.VectorSubcoreMesh(core_axis_name="c", subcore_axis_name="s")

    @functools.partial(
        pl.kernel,
        out_type=jax.ShapeDtypeStruct((e, d), jnp.float32),
        mesh=mesh,
        scratch_types=[
            pltpu.VMEM((chunk,), jnp.int32),
            pltpu.VMEM((chunk, d), jnp.float32),
            pltpu.SemaphoreType.DMA,
        ],
    )
    def gather_k(t_hbm, idx_hbm, out_hbm, idx_v, rows_v, sem):
        wid = lax.axis_index("s") * nc + lax.axis_index("c")
        base = wid * epw

        def body(i, carry):
            off = base + i * chunk
            pltpu.sync_copy(idx_hbm.at[pl.ds(off, chunk)], idx_v)
            pltpu.async_copy(t_hbm.at[idx_v], rows_v, sem).wait()
            pltpu.sync_copy(rows_v, out_hbm.at[pl.ds(off, chunk)])
            return carry

        lax.fori_loop(0, nchunks, body, 0)

    return gather_k(table, flat_idx)


def _ln(x, g, b, eps=1e-5):
    mu = jnp.mean(x, axis=-1, keepdims=True)
    var = jnp.mean((x - mu) ** 2, axis=-1, keepdims=True)
    return (x - mu) / jnp.sqrt(var + eps) * g + b


def _gelu(x):
    return 0.5 * x * (1.0 + lax.erf(x * 0.7071067811865476))


def _main_body(nh_ref, eh_ref, tg_ref, ar_ref, w1a, b1r, w1c, w2, b2r, w3, b3r,
               wf1, bf1r, wf2, bf2r, g1r, be1r, g2r, be2r, out_ref):
    blk, k, h = eh_ref.shape
    f32 = jnp.float32
    nh = nh_ref[:]
    a = jnp.dot(nh, w1a[:], preferred_element_type=f32) + b1r[:]
    eh = eh_ref[:].reshape(blk * k, h)
    pre = jnp.dot(eh, w1c[:], preferred_element_type=f32).reshape(blk, k, h)
    tg = tg_ref[:]
    bg = tg[:, :, :h]
    dg = tg[:, :, h:]
    arb = lax.broadcast_in_dim(ar_ref[:], (blk, k, h), (0, 1))
    x = pre + a[:, None, :] + bg + arb * dg
    x = _gelu(x)
    x2 = _gelu(jnp.dot(x.reshape(blk * k, h), w2[:], preferred_element_type=f32) + b2r[:])
    s = jnp.sum(x2.reshape(blk, k, h), axis=1)
    agg = jnp.dot(s, w3[:], preferred_element_type=f32) + k * b3r[:]
    nh1 = _ln(nh + agg, g1r[:], be1r[:])
    ff1 = _gelu(jnp.dot(nh1, wf1[:], preferred_element_type=f32) + bf1r[:])
    ff = jnp.dot(ff1, wf2[:], preferred_element_type=f32) + bf2r[:]
    out_ref[:] = _ln(nh1 + ff, g2r[:], be2r[:])


def _main(node_h, edge_h, tg3, ar_mask, w1a, b1r, w1c, w2, b2r, w3, b3r,
          wf1, bf1r, wf2, bf2r, g1r, be1r, g2r, be2r, blk=200, interpret=False):
    res, k, h = edge_h.shape
    nb = res // blk
    full = lambda s: pl.BlockSpec(s, lambda i: tuple(0 for _ in s))
    return pl.pallas_call(
        _main_body,
        grid=(nb,),
        in_specs=[
            pl.BlockSpec((blk, h), lambda i: (i, 0)),
            pl.BlockSpec((blk, k, h), lambda i: (i, 0, 0)),
            pl.BlockSpec((blk, k, 2 * h), lambda i: (i, 0, 0)),
            pl.BlockSpec((blk, k), lambda i: (i, 0)),
            full((h, h)), full((1, h)),
            full((h, h)), full((h, h)), full((1, h)),
            full((h, h)), full((1, h)),
            full((h, 4 * h)), full((1, 4 * h)),
            full((4 * h, h)), full((1, h)),
            full((1, h)), full((1, h)), full((1, h)), full((1, h)),
        ],
        out_specs=pl.BlockSpec((blk, h), lambda i: (i, 0)),
        out_shape=jax.ShapeDtypeStruct((res, h), jnp.float32),
        interpret=interpret,
    )(node_h, edge_h, tg3, ar_mask, w1a, b1r, w1c, w2, b2r, w3, b3r,
      wf1, bf1r, wf2, bf2r, g1r, be1r, g2r, be2r)


def kernel(node_h, edge_h, edge_idx, seq_emb, ar_mask, W1, b1, W2, b2, W3, b3,
           Wf1, bf1, Wf2, bf2, g1, be1, g2, be2):
    res, h = node_h.shape
    k = edge_idx.shape[1]
    w1a, w1b, w1c, w1d = W1[:h], W1[h:2 * h], W1[2 * h:3 * h], W1[3 * h:]
    table = _make_table(node_h, seq_emb, w1b, w1d)
    tg = _sc_gather(table, edge_idx.reshape(-1))
    r1 = lambda v: v.reshape(1, -1)
    return _main(node_h, edge_h, tg.reshape(res, k, 2 * h), ar_mask,
                 w1a, r1(b1), w1c, W2, r1(b2), W3, r1(b3),
                 Wf1, r1(bf1), Wf2, r1(bf2), r1(g1), r1(be1), r1(g2), r1(be2))


# speedup vs baseline: 6.2702x; 1.3357x over previous
import functools

import jax
import jax.numpy as jnp
from jax import lax
from jax.experimental import pallas as pl
from jax.experimental.pallas import tpu as pltpu
from jax.experimental.pallas import tpu_sc as plsc


def _table_body(nh_ref, se_ref, w1b_ref, w1d_ref, t_ref):
    h = nh_ref.shape[1]
    t_ref[:, :h] = jnp.dot(nh_ref[:], w1b_ref[:], preferred_element_type=jnp.float32)
    t_ref[:, h:] = jnp.dot(se_ref[:], w1d_ref[:], preferred_element_type=jnp.float32)


def _make_table(node_h, seq_emb, w1b, w1d, blk=1000):
    res, h = node_h.shape
    nb = res // blk
    return pl.pallas_call(
        _table_body,
        grid=(nb,),
        in_specs=[
            pl.BlockSpec((blk, h), lambda i: (i, 0)),
            pl.BlockSpec((blk, h), lambda i: (i, 0)),
            pl.BlockSpec((h, h), lambda i: (0, 0)),
            pl.BlockSpec((h, h), lambda i: (0, 0)),
        ],
        out_specs=pl.BlockSpec((blk, 2 * h), lambda i: (i, 0)),
        out_shape=jax.ShapeDtypeStruct((res, 2 * h), jnp.float32),
    )(node_h, seq_emb, w1b, w1d)


def _sc_gather(table, flat_idx, chunk=80):
    res, d = table.shape
    e = flat_idx.shape[0]
    info = plsc.get_sparse_core_info()
    nc, ns = info.num_cores, info.num_subcores
    nw = nc * ns
    epw = e // nw
    nchunks = epw // chunk
    assert nchunks % 2 == 1 and chunk % 8 == 0
    mesh = plsc.VectorSubcoreMesh(core_axis_name="c", subcore_axis_name="s")

    @functools.partial(
        pl.kernel,
        out_type=jax.ShapeDtypeStruct((e, d), jnp.float32),
        mesh=mesh,
        scratch_types=[
            pltpu.VMEM((epw,), jnp.int32),
            pltpu.VMEM((chunk, d), jnp.float32),
            pltpu.VMEM((chunk, d), jnp.float32),
            pltpu.SemaphoreType.DMA,
            pltpu.SemaphoreType.DMA,
            pltpu.SemaphoreType.DMA,
            pltpu.SemaphoreType.DMA,
        ],
    )
    def gather_k(t_hbm, idx_hbm, out_hbm, idx_all, rows0, rows1, g0, g1, s0, s1):
        wid = lax.axis_index("s") * nc + lax.axis_index("c")
        base = wid * epw
        rows = (rows0, rows1)
        gsem = (g0, g1)
        ssem = (s0, s1)

        def start_gather(j, b):
            pltpu.async_copy(t_hbm.at[idx_all.at[pl.ds(j * chunk, chunk)]],
                             rows[b], gsem[b])

        def wait_gather(b):
            pltpu.make_async_copy(t_hbm.at[idx_all.at[pl.ds(0, chunk)]],
                                  rows[b], gsem[b]).wait()

        def start_store(j, b):
            pltpu.async_copy(rows[b], out_hbm.at[pl.ds(base + j * chunk, chunk)],
                             ssem[b])

        def wait_store(b):
            pltpu.make_async_copy(rows[b], out_hbm.at[pl.ds(base, chunk)],
                                  ssem[b]).wait()

        pltpu.sync_copy(idx_hbm.at[pl.ds(base, epw)], idx_all)
        start_gather(0, 0)
        start_store(1, 1)

        def pair(p, carry):
            j0 = 2 * p
            for s in range(2):
                j = j0 + s
                b = s
                nb = 1 - s
                wait_store(nb)
                start_gather_j1 = j + 1
                pltpu.async_copy(
                    t_hbm.at[idx_all.at[pl.ds(start_gather_j1 * chunk, chunk)]],
                    rows[nb], gsem[nb])
                wait_gather(b)
                start_store(j, b)
            return carry

        lax.fori_loop(0, (nchunks - 1) // 2, pair, 0)
        jl = nchunks - 1
        wait_gather(0)
        start_store(jl, 0)
        wait_store(1)
        wait_store(0)

    return gather_k(table, flat_idx)


def _ln(x, g, b, eps=1e-5):
    mu = jnp.mean(x, axis=-1, keepdims=True)
    var = jnp.mean((x - mu) ** 2, axis=-1, keepdims=True)
    return (x - mu) / jnp.sqrt(var + eps) * g + b


def _gelu(x):
    return 0.5 * x * (1.0 + lax.erf(x * 0.7071067811865476))


def _main_body(nh_ref, eh_ref, tg_ref, ar_ref, w1a, b1r, w1c, w2, b2r, w3, b3r,
               wf1, bf1r, wf2, bf2r, g1r, be1r, g2r, be2r, out_ref):
    blk, k, h = eh_ref.shape
    f32 = jnp.float32
    nh = nh_ref[:]
    a = jnp.dot(nh, w1a[:], preferred_element_type=f32) + b1r[:]
    eh = eh_ref[:].reshape(blk * k, h)
    pre = jnp.dot(eh, w1c[:], preferred_element_type=f32).reshape(blk, k, h)
    tg = tg_ref[:]
    bg = tg[:, :, :h]
    dg = tg[:, :, h:]
    arb = lax.broadcast_in_dim(ar_ref[:], (blk, k, h), (0, 1))
    x = pre + a[:, None, :] + bg + arb * dg
    x = _gelu(x)
    x2 = _gelu(jnp.dot(x.reshape(blk * k, h), w2[:], preferred_element_type=f32) + b2r[:])
    s = jnp.sum(x2.reshape(blk, k, h), axis=1)
    agg = jnp.dot(s, w3[:], preferred_element_type=f32) + k * b3r[:]
    nh1 = _ln(nh + agg, g1r[:], be1r[:])
    ff1 = _gelu(jnp.dot(nh1, wf1[:], preferred_element_type=f32) + bf1r[:])
    ff = jnp.dot(ff1, wf2[:], preferred_element_type=f32) + bf2r[:]
    out_ref[:] = _ln(nh1 + ff, g2r[:], be2r[:])


def _main(node_h, edge_h, tg3, ar_mask, w1a, b1r, w1c, w2, b2r, w3, b3r,
          wf1, bf1r, wf2, bf2r, g1r, be1r, g2r, be2r, blk=200, interpret=False):
    res, k, h = edge_h.shape
    nb = res // blk
    full = lambda s: pl.BlockSpec(s, lambda i: tuple(0 for _ in s))
    return pl.pallas_call(
        _main_body,
        grid=(nb,),
        in_specs=[
            pl.BlockSpec((blk, h), lambda i: (i, 0)),
            pl.BlockSpec((blk, k, h), lambda i: (i, 0, 0)),
            pl.BlockSpec((blk, k, 2 * h), lambda i: (i, 0, 0)),
            pl.BlockSpec((blk, k), lambda i: (i, 0)),
            full((h, h)), full((1, h)),
            full((h, h)), full((h, h)), full((1, h)),
            full((h, h)), full((1, h)),
            full((h, 4 * h)), full((1, 4 * h)),
            full((4 * h, h)), full((1, h)),
            full((1, h)), full((1, h)), full((1, h)), full((1, h)),
        ],
        out_specs=pl.BlockSpec((blk, h), lambda i: (i, 0)),
        out_shape=jax.ShapeDtypeStruct((res, h), jnp.float32),
        interpret=interpret,
    )(node_h, edge_h, tg3, ar_mask, w1a, b1r, w1c, w2, b2r, w3, b3r,
      wf1, bf1r, wf2, bf2r, g1r, be1r, g2r, be2r)


def kernel(node_h, edge_h, edge_idx, seq_emb, ar_mask, W1, b1, W2, b2, W3, b3,
           Wf1, bf1, Wf2, bf2, g1, be1, g2, be2):
    res, h = node_h.shape
    k = edge_idx.shape[1]
    w1a, w1b, w1c, w1d = W1[:h], W1[h:2 * h], W1[2 * h:3 * h], W1[3 * h:]
    table = _make_table(node_h, seq_emb, w1b, w1d)
    tg = _sc_gather(table, edge_idx.reshape(-1))
    r1 = lambda v: v.reshape(1, -1)
    return _main(node_h, edge_h, tg.reshape(res, k, 2 * h), ar_mask,
                 w1a, r1(b1), w1c, W2, r1(b2), W3, r1(b3),
                 Wf1, r1(bf1), Wf2, r1(bf2), r1(g1), r1(be1), r1(g2), r1(be2))


# speedup vs baseline: 8.4483x; 1.3474x over previous
import functools

import jax
import jax.numpy as jnp
from jax import lax
from jax.experimental import pallas as pl
from jax.experimental.pallas import tpu as pltpu
from jax.experimental.pallas import tpu_sc as plsc


def _table_body(nh_ref, se_ref, w1b_ref, w1d_ref, t_ref):
    bf = jnp.dot(nh_ref[:], w1b_ref[:], preferred_element_type=jnp.float32)
    df = jnp.dot(se_ref[:], w1d_ref[:], preferred_element_type=jnp.float32)
    bw = (lax.bitcast_convert_type(bf, jnp.uint32) + 0x8000) >> 16
    dw = ((lax.bitcast_convert_type(df, jnp.uint32) + 0x8000) >> 16) << 16
    t_ref[...] = dw | bw


def _make_table(node_h, seq_emb, w1b, w1d, blk=1000):
    res, h = node_h.shape
    nb = res // blk
    return pl.pallas_call(
        _table_body,
        grid=(nb,),
        in_specs=[
            pl.BlockSpec((blk, h), lambda i: (i, 0)),
            pl.BlockSpec((blk, h), lambda i: (i, 0)),
            pl.BlockSpec((h, h), lambda i: (0, 0)),
            pl.BlockSpec((h, h), lambda i: (0, 0)),
        ],
        out_specs=pl.BlockSpec((blk, h), lambda i: (i, 0)),
        out_shape=jax.ShapeDtypeStruct((res, h), jnp.uint32),
    )(node_h, seq_emb, w1b, w1d)


def _sc_gather(table, flat_idx, chunk=80):
    res, d = table.shape
    e = flat_idx.shape[0]
    info = plsc.get_sparse_core_info()
    nc, ns = info.num_cores, info.num_subcores
    nw = nc * ns
    epw = e // nw
    nchunks = epw // chunk
    assert nchunks % 2 == 1 and chunk % 8 == 0
    mesh = plsc.VectorSubcoreMesh(core_axis_name="c", subcore_axis_name="s")

    @functools.partial(
        pl.kernel,
        out_type=jax.ShapeDtypeStruct((e, d), table.dtype),
        mesh=mesh,
        scratch_types=[
            pltpu.VMEM((epw,), jnp.int32),
            pltpu.VMEM((chunk, d), table.dtype),
            pltpu.VMEM((chunk, d), table.dtype),
            pltpu.SemaphoreType.DMA,
            pltpu.SemaphoreType.DMA,
            pltpu.SemaphoreType.DMA,
            pltpu.SemaphoreType.DMA,
        ],
    )
    def gather_k(t_hbm, idx_hbm, out_hbm, idx_all, rows0, rows1, g0, g1, s0, s1):
        wid = lax.axis_index("s") * nc + lax.axis_index("c")
        base = wid * epw
        rows = (rows0, rows1)
        gsem = (g0, g1)
        ssem = (s0, s1)

        def start_gather(j, b):
            pltpu.async_copy(t_hbm.at[idx_all.at[pl.ds(j * chunk, chunk)]],
                             rows[b], gsem[b])

        def wait_gather(b):
            pltpu.make_async_copy(t_hbm.at[idx_all.at[pl.ds(0, chunk)]],
                                  rows[b], gsem[b]).wait()

        def start_store(j, b):
            pltpu.async_copy(rows[b], out_hbm.at[pl.ds(base + j * chunk, chunk)],
                             ssem[b])

        def wait_store(b):
            pltpu.make_async_copy(rows[b], out_hbm.at[pl.ds(base, chunk)],
                                  ssem[b]).wait()

        pltpu.sync_copy(idx_hbm.at[pl.ds(base, epw)], idx_all)
        start_gather(0, 0)
        start_store(1, 1)

        def pair(p, carry):
            j0 = 2 * p
            for s in range(2):
                j = j0 + s
                b = s
                nb = 1 - s
                wait_store(nb)
                start_gather_j1 = j + 1
                pltpu.async_copy(
                    t_hbm.at[idx_all.at[pl.ds(start_gather_j1 * chunk, chunk)]],
                    rows[nb], gsem[nb])
                wait_gather(b)
                start_store(j, b)
            return carry

        lax.fori_loop(0, (nchunks - 1) // 2, pair, 0)
        jl = nchunks - 1
        wait_gather(0)
        start_store(jl, 0)
        wait_store(1)
        wait_store(0)

    return gather_k(table, flat_idx)


def _ln(x, g, b, eps=1e-5):
    mu = jnp.mean(x, axis=-1, keepdims=True)
    var = jnp.mean((x - mu) ** 2, axis=-1, keepdims=True)
    return (x - mu) / jnp.sqrt(var + eps) * g + b


def _gelu(x):
    return 0.5 * x * (1.0 + lax.erf(x * 0.7071067811865476))


def _main_body(nh_ref, eh_ref, tg_ref, ar_ref, w1a, b1r, w1c, w2, b2r, w3, b3r,
               wf1, bf1r, wf2, bf2r, g1r, be1r, g2r, be2r, out_ref):
    blk, k, h = eh_ref.shape
    f32 = jnp.float32
    nh = nh_ref[:]
    a = jnp.dot(nh, w1a[:], preferred_element_type=f32) + b1r[:]
    eh = eh_ref[:].reshape(blk * k, h)
    pre = jnp.dot(eh, w1c[:], preferred_element_type=f32).reshape(blk, k, h)
    tg = tg_ref[:]
    bg = lax.bitcast_convert_type(tg << 16, jnp.float32)
    dg = lax.bitcast_convert_type(tg & jnp.uint32(0xFFFF0000), jnp.float32)
    arb = lax.broadcast_in_dim(ar_ref[:], (blk, k, h), (0, 1))
    x = pre + a[:, None, :] + bg + arb * dg
    x = _gelu(x)
    x2 = _gelu(jnp.dot(x.reshape(blk * k, h), w2[:], preferred_element_type=f32) + b2r[:])
    s = jnp.sum(x2.reshape(blk, k, h), axis=1)
    agg = jnp.dot(s, w3[:], preferred_element_type=f32) + k * b3r[:]
    nh1 = _ln(nh + agg, g1r[:], be1r[:])
    ff1 = _gelu(jnp.dot(nh1, wf1[:], preferred_element_type=f32) + bf1r[:])
    ff = jnp.dot(ff1, wf2[:], preferred_element_type=f32) + bf2r[:]
    out_ref[:] = _ln(nh1 + ff, g2r[:], be2r[:])


def _main(node_h, edge_h, tg3, ar_mask, w1a, b1r, w1c, w2, b2r, w3, b3r,
          wf1, bf1r, wf2, bf2r, g1r, be1r, g2r, be2r, blk=200, interpret=False):
    res, k, h = edge_h.shape
    nb = res // blk
    full = lambda s: pl.BlockSpec(s, lambda i: tuple(0 for _ in s))
    return pl.pallas_call(
        _main_body,
        grid=(nb,),
        in_specs=[
            pl.BlockSpec((blk, h), lambda i: (i, 0)),
            pl.BlockSpec((blk, k, h), lambda i: (i, 0, 0)),
            pl.BlockSpec((blk, k, h), lambda i: (i, 0, 0)),
            pl.BlockSpec((blk, k), lambda i: (i, 0)),
            full((h, h)), full((1, h)),
            full((h, h)), full((h, h)), full((1, h)),
            full((h, h)), full((1, h)),
            full((h, 4 * h)), full((1, 4 * h)),
            full((4 * h, h)), full((1, h)),
            full((1, h)), full((1, h)), full((1, h)), full((1, h)),
        ],
        out_specs=pl.BlockSpec((blk, h), lambda i: (i, 0)),
        out_shape=jax.ShapeDtypeStruct((res, h), jnp.float32),
        interpret=interpret,
    )(node_h, edge_h, tg3, ar_mask, w1a, b1r, w1c, w2, b2r, w3, b3r,
      wf1, bf1r, wf2, bf2r, g1r, be1r, g2r, be2r)


def kernel(node_h, edge_h, edge_idx, seq_emb, ar_mask, W1, b1, W2, b2, W3, b3,
           Wf1, bf1, Wf2, bf2, g1, be1, g2, be2):
    res, h = node_h.shape
    k = edge_idx.shape[1]
    w1a, w1b, w1c, w1d = W1[:h], W1[h:2 * h], W1[2 * h:3 * h], W1[3 * h:]
    table = _make_table(node_h, seq_emb, w1b, w1d)
    tg = _sc_gather(table, edge_idx.reshape(-1))
    r1 = lambda v: v.reshape(1, -1)
    return _main(node_h, edge_h, tg.reshape(res, k, h), ar_mask,
                 w1a, r1(b1), w1c, W2, r1(b2), W3, r1(b3),
                 Wf1, r1(bf1), Wf2, r1(bf2), r1(g1), r1(be1), r1(g2), r1(be2))
